# baseline (speedup 1.0000x reference)
import functools
import numpy as np
import jax
import jax.numpy as jnp
from jax import lax
from jax.experimental import pallas as pl
from jax.experimental.pallas import tpu as pltpu
from jax.experimental.pallas import tpu_sc as plsc

B, E, H, W = 8, 128, 128, 128
NQ = H * W
HEADS, PTS, HD = 8, 4, 16
QBLK = 256
QTOT = B * NQ
NC, NS = 2, 16
NW = NC * NS
CH = 8


_DN_T = (((0,), (0,)), ((), ()))


def _prep_body(x_ref, wv_ref, bv_ref, wx_ref, bx_ref, wy_ref, by_ref,
               wa_ref, ba_ref, kxi_ref, kyi_ref, hl_ref,
               ax_ref, bxk_ref, ay_ref, byk_ref,
               val_ref, idx_ref, w_ref):
    b = pl.program_id(0)
    g = pl.program_id(1)
    xb = x_ref[0]
    val_ref[...] = lax.dot_general(xb, wv_ref[...], _DN_T,
                                 preferred_element_type=jnp.float32) + bv_ref[0]
    offx = lax.dot_general(xb, wx_ref[...], _DN_T,
                           preferred_element_type=jnp.float32) + bx_ref[0]
    offy = lax.dot_general(xb, wy_ref[...], _DN_T,
                           preferred_element_type=jnp.float32) + by_ref[0]
    al = lax.dot_general(xb, wa_ref[...], _DN_T,
                         preferred_element_type=jnp.float32) + ba_ref[0]
    m = al
    for s in (32, 64, 96):
        m = jnp.maximum(m, pltpu.roll(al, s, 1))
    ex = jnp.exp(al - m)
    ssum = ex
    for s in (32, 64, 96):
        ssum = ssum + pltpu.roll(ex, s, 1)
    attn = ex / ssum
    i0 = lax.broadcasted_iota(jnp.int32, (QBLK, 128), 0)
    col = i0 & (W - 1)
    rowf = (jnp.where(i0 < W, np.float32(0.0), np.float32(H / (H - 1)))
            + g.astype(jnp.float32) * np.float32((QBLK // W) * H / (H - 1)))
    gx = col.astype(jnp.float32) * np.float32(W / (W - 1)) + offx
    gy = rowf + offy
    x0f = jnp.floor(gx)
    y0f = jnp.floor(gy)
    fx = gx - x0f
    fy = gy - y0f
    xi = x0f.astype(jnp.int32) + kxi_ref[0]
    yi = y0f.astype(jnp.int32) + kyi_ref[0]
    wx = ax_ref[0] + bxk_ref[0] * fx
    wy = ay_ref[0] + byk_ref[0] * fy
    ux = xi.astype(jnp.uint32) < np.uint32(W)
    uy = yi.astype(jnp.uint32) < np.uint32(H)
    valid = ux & uy
    spat = ((yi & (H - 1)) << 7) | (xi & (W - 1))
    idx_ref[...] = (spat << 3) + (b * np.int32(NQ * HEADS) + hl_ref[0])
    w_ref[...] = (attn * wx) * (wy * jnp.where(valid, np.float32(1.0),
                                             np.float32(0.0)))


def _post_body(s_ref, x_ref, wo_ref, bo_ref, out_ref):
    y = lax.dot_general(wo_ref[...], s_ref[...], (((1,), (1,)), ((), ())),
                        preferred_element_type=jnp.float32)
    out_ref[0] = (y + bo_ref[...]) + x_ref[0]


def _sc_gather_body(qtot, table_hbm, idx_hbm, w_hbm, out_hbm,
                    idx_v, w_v, rows_v, out_v,
                    sem_in0, sem_in1, sem_g0, sem_g1, sem_out0, sem_out1):
    cid = lax.axis_index("c")
    sid = lax.axis_index("s")
    wid = sid * NC + cid
    per_w = qtot // NW
    nchunks = per_w // CH
    wbase = wid * per_w
    sem_in = (sem_in0, sem_in1)
    sem_g = (sem_g0, sem_g1)
    sem_out = (sem_out0, sem_out1)

    def stage_idx(ci, s):
        pltpu.async_copy(idx_hbm.at[pl.ds(wbase + ci * CH, CH)],
                         idx_v.at[s], sem_in[s])

    def stage_w(ci, s):
        pltpu.async_copy(w_hbm.at[pl.ds(wbase + ci * CH, CH)],
                         w_v.at[s], sem_in[s])

    def wait_in(s):
        pltpu.make_async_copy(idx_hbm.at[pl.ds(0, CH)], idx_v.at[s],
                              sem_in[s]).wait()
        pltpu.make_async_copy(w_hbm.at[pl.ds(0, CH)], w_v.at[s],
                              sem_in[s]).wait()

    def fire_gathers(s):
        for qi in range(CH):
            pltpu.async_copy(table_hbm.at[idx_v.at[s, qi]],
                             rows_v.at[s, qi], sem_g[s])

    def drain_gathers(s):
        for qi in range(CH):
            pltpu.make_async_copy(table_hbm.at[idx_v.at[s, qi]],
                                  rows_v.at[s, qi], sem_g[s]).wait()

    def compute(s):
        @plsc.parallel_loop(0, CH, unroll=2)
        def _(qi):
            wv = [w_v[s, qi, pl.ds(g * 16, 16)] for g in range(8)]
            for hh in range(HEADS):
                acc = wv[0][hh] * rows_v[s, qi, hh, :]
                for mm in range(1, 16):
                    j = mm * 8 + hh
                    acc = acc + wv[mm // 2][(mm % 2) * 8 + hh] * rows_v[s, qi, j, :]
                out_v[s, qi, pl.ds(hh * HD, HD)] = acc

    def start_out(ci, s):
        pltpu.async_copy(out_v.at[s],
                         out_hbm.at[pl.ds(wbase + ci * CH, CH)], sem_out[s])

    def wait_out(s):
        pltpu.make_async_copy(out_v.at[s], out_hbm.at[pl.ds(0, CH)],
                              sem_out[s]).wait()

    stage_idx(0, 0)
    stage_w(0, 0)
    stage_idx(1, 1)
    stage_w(1, 1)
    wait_in(0)
    fire_gathers(0)

    def half(ci, s):
        o = 1 - s

        @pl.when(ci + 1 < nchunks)
        def _():
            wait_in(o)
            fire_gathers(o)

        drain_gathers(s)

        @pl.when(ci + 2 < nchunks)
        def _():
            stage_idx(ci + 2, s)

        @pl.when(ci >= 2)
        def _():
            wait_out(s)

        compute(s)
        start_out(ci, s)

        @pl.when(ci + 2 < nchunks)
        def _():
            stage_w(ci + 2, s)

    def pair(i, carry):
        half(2 * i, 0)
        half(2 * i + 1, 1)
        return carry

    lax.fori_loop(0, nchunks // 2, pair, 0)
    wait_out(0)
    wait_out(1)


def _sc_gather(table, idx, w):
    qtot = idx.shape[0]
    mesh = plsc.VectorSubcoreMesh(core_axis_name="c", subcore_axis_name="s")
    return pl.kernel(
        functools.partial(_sc_gather_body, qtot),
        out_type=jax.ShapeDtypeStruct((qtot, HEADS * HD), jnp.float32),
        mesh=mesh,
        scratch_types=[
            pltpu.VMEM((2, CH, 128), jnp.int32),
            pltpu.VMEM((2, CH, 128), jnp.float32),
            pltpu.VMEM((2, CH, 128, HD), jnp.float32),
            pltpu.VMEM((2, CH, 128), jnp.float32),
            pltpu.SemaphoreType.DMA,
            pltpu.SemaphoreType.DMA,
            pltpu.SemaphoreType.DMA,
            pltpu.SemaphoreType.DMA,
            pltpu.SemaphoreType.DMA,
            pltpu.SemaphoreType.DMA,
        ],
        compiler_params=pltpu.CompilerParams(use_tc_tiling_on_sc=False),
    )(table, idx, w)


def kernel(x, W_off, b_off, W_attn, b_attn, W_val, b_val, W_out, b_out):
    x2 = x.reshape(B, E, NQ)
    ell = np.arange(128)
    p = ell >> 5
    k = (ell >> 3) & 3
    h = ell & 7
    rx = h * (PTS * 2) + p * 2
    Wx = W_off[rx].T
    bx = b_off[rx].reshape(1, 128) - np.float32(0.5)
    Wy = W_off[rx + 1].T
    by = b_off[rx + 1].reshape(1, 128) - np.float32(0.5)
    ra = h * PTS + p
    Wa = W_attn[ra].T
    ba = b_attn[ra].reshape(1, 128)
    kx = (k & 1).astype(np.int32)
    ky = (k >> 1).astype(np.int32)
    kxi = jnp.asarray(kx.reshape(1, 128))
    kyi = jnp.asarray(ky.reshape(1, 128))
    hl = jnp.asarray(h.astype(np.int32).reshape(1, 128))
    ax = jnp.asarray((1.0 - kx).astype(np.float32).reshape(1, 128))
    bxk = jnp.asarray((2.0 * kx - 1.0).astype(np.float32).reshape(1, 128))
    ay = jnp.asarray((1.0 - ky).astype(np.float32).reshape(1, 128))
    byk = jnp.asarray((2.0 * ky - 1.0).astype(np.float32).reshape(1, 128))

    nhalf = 8
    bh = B // nhalf
    qtot_h = bh * NQ
    gpb = NQ // QBLK
    grid = (bh, gpb)
    wspec = pl.BlockSpec((E, 128), lambda b_, g_: (0, 0))
    bspec = pl.BlockSpec((1, 128), lambda b_, g_: (0, 0))
    qspec = pl.BlockSpec((QBLK, 128), lambda b_, g_: (b_ * gpb + g_, 0))
    wv_t = W_val.T
    bv = b_val.reshape(1, 128)
    bo = b_out.reshape(128, 1)

    outs = []
    for hf in range(nhalf):
        xh = lax.slice_in_dim(x2, hf * bh, (hf + 1) * bh, axis=0)
        val, idx, w = pl.pallas_call(
            _prep_body,
            grid=grid,
            in_specs=[
                pl.BlockSpec((1, E, QBLK), lambda b_, g_: (b_, 0, g_)),
                wspec, bspec, wspec, bspec, wspec, bspec, wspec, bspec,
                bspec, bspec, bspec, bspec, bspec, bspec, bspec,
            ],
            out_specs=[qspec, qspec, qspec],
            out_shape=[
                jax.ShapeDtypeStruct((qtot_h, E), jnp.float32),
                jax.ShapeDtypeStruct((qtot_h, 128), jnp.int32),
                jax.ShapeDtypeStruct((qtot_h, 128), jnp.float32),
            ],
        )(xh, wv_t, bv, Wx, bx, Wy, by, Wa, ba,
          kxi, kyi, hl, ax, bxk, ay, byk)

        table = val.reshape(qtot_h * HEADS, HD)
        sampled = _sc_gather(table, idx, w)

        out_h = pl.pallas_call(
            _post_body,
            grid=grid,
            in_specs=[
                qspec,
                pl.BlockSpec((1, E, QBLK), lambda b_, g_: (b_, 0, g_)),
                wspec,
                pl.BlockSpec((E, 1), lambda b_, g_: (0, 0)),
            ],
            out_specs=pl.BlockSpec((1, E, QBLK), lambda b_, g_: (b_, 0, g_)),
            out_shape=jax.ShapeDtypeStruct((bh, E, NQ), jnp.float32),
        )(sampled, xh, W_out, bo)
        outs.append(out_h)
    out = jnp.concatenate(outs, axis=0)
    return out.reshape(B, E, H, W)


# speedup vs baseline: 1.0620x; 1.0620x over previous
import functools
import numpy as np
import jax
import jax.numpy as jnp
from jax import lax
from jax.experimental import pallas as pl
from jax.experimental.pallas import tpu as pltpu
from jax.experimental.pallas import tpu_sc as plsc

B, E, H, W = 8, 128, 128, 128
NQ = H * W
HEADS, PTS, HD = 8, 4, 16
QBLK = 256
QTOT = B * NQ
NC, NS = 2, 16
NW = NC * NS
CH = 16


_DN_T = (((0,), (0,)), ((), ()))


def _prep_body(x_ref, wv_ref, bv_ref, wx_ref, bx_ref, wy_ref, by_ref,
               wa_ref, ba_ref, kxi_ref, kyi_ref, hl_ref,
               ax_ref, bxk_ref, ay_ref, byk_ref,
               val_ref, idx_ref, w_ref):
    b = pl.program_id(0)
    g = pl.program_id(1)
    xb = x_ref[0]
    val_ref[...] = lax.dot_general(xb, wv_ref[...], _DN_T,
                                 preferred_element_type=jnp.float32) + bv_ref[0]
    offx = lax.dot_general(xb, wx_ref[...], _DN_T,
                           preferred_element_type=jnp.float32) + bx_ref[0]
    offy = lax.dot_general(xb, wy_ref[...], _DN_T,
                           preferred_element_type=jnp.float32) + by_ref[0]
    al = lax.dot_general(xb, wa_ref[...], _DN_T,
                         preferred_element_type=jnp.float32) + ba_ref[0]
    m = al
    for s in (32, 64, 96):
        m = jnp.maximum(m, pltpu.roll(al, s, 1))
    ex = jnp.exp(al - m)
    ssum = ex
    for s in (32, 64, 96):
        ssum = ssum + pltpu.roll(ex, s, 1)
    attn = ex / ssum
    i0 = lax.broadcasted_iota(jnp.int32, (QBLK, 128), 0)
    col = i0 & (W - 1)
    rowf = (jnp.where(i0 < W, np.float32(0.0), np.float32(H / (H - 1)))
            + g.astype(jnp.float32) * np.float32((QBLK // W) * H / (H - 1)))
    gx = col.astype(jnp.float32) * np.float32(W / (W - 1)) + offx
    gy = rowf + offy
    x0f = jnp.floor(gx)
    y0f = jnp.floor(gy)
    fx = gx - x0f
    fy = gy - y0f
    xi = x0f.astype(jnp.int32) + kxi_ref[0]
    yi = y0f.astype(jnp.int32) + kyi_ref[0]
    wx = ax_ref[0] + bxk_ref[0] * fx
    wy = ay_ref[0] + byk_ref[0] * fy
    ux = xi.astype(jnp.uint32) < np.uint32(W)
    uy = yi.astype(jnp.uint32) < np.uint32(H)
    valid = ux & uy
    spat = ((yi & (H - 1)) << 7) | (xi & (W - 1))
    idx_ref[...] = (spat << 3) + (b * np.int32(NQ * HEADS) + hl_ref[0])
    w_ref[...] = (attn * wx) * (wy * jnp.where(valid, np.float32(1.0),
                                             np.float32(0.0)))


def _post_body(s_ref, x_ref, wo_ref, bo_ref, out_ref):
    y = lax.dot_general(wo_ref[...], s_ref[...], (((1,), (1,)), ((), ())),
                        preferred_element_type=jnp.float32)
    out_ref[0] = (y + bo_ref[...]) + x_ref[0]


def _sc_gather_body(qtot, table_hbm, idx_hbm, w_hbm, out_hbm,
                    idx_v, w_v, rows_v, out_v,
                    sem_in0, sem_in1, sem_g0, sem_g1, sem_out0, sem_out1):
    cid = lax.axis_index("c")
    sid = lax.axis_index("s")
    wid = sid * NC + cid
    per_w = qtot // NW
    nchunks = per_w // CH
    wbase = wid * per_w
    sem_in = (sem_in0, sem_in1)
    sem_g = (sem_g0, sem_g1)
    sem_out = (sem_out0, sem_out1)

    def stage_idx(ci, s):
        pltpu.async_copy(idx_hbm.at[pl.ds(wbase + ci * CH, CH)],
                         idx_v.at[s], sem_in[s])

    def stage_w(ci, s):
        pltpu.async_copy(w_hbm.at[pl.ds(wbase + ci * CH, CH)],
                         w_v.at[s], sem_in[s])

    def wait_in(s):
        pltpu.make_async_copy(idx_hbm.at[pl.ds(0, CH)], idx_v.at[s],
                              sem_in[s]).wait()
        pltpu.make_async_copy(w_hbm.at[pl.ds(0, CH)], w_v.at[s],
                              sem_in[s]).wait()

    def fire_gathers(s):
        for qi in range(CH):
            pltpu.async_copy(table_hbm.at[idx_v.at[s, qi]],
                             rows_v.at[s, qi], sem_g[s])

    def drain_gathers(s):
        for qi in range(CH):
            pltpu.make_async_copy(table_hbm.at[idx_v.at[s, qi]],
                                  rows_v.at[s, qi], sem_g[s]).wait()

    def compute(s):
        @plsc.parallel_loop(0, CH, unroll=4)
        def _(qi):
            wv = [w_v[s, qi, pl.ds(g * 16, 16)] for g in range(8)]
            for hh in range(HEADS):
                acc = wv[0][hh] * rows_v[s, qi, hh, :]
                for mm in range(1, 16):
                    j = mm * 8 + hh
                    acc = acc + wv[mm // 2][(mm % 2) * 8 + hh] * rows_v[s, qi, j, :]
                out_v[s, qi, pl.ds(hh * HD, HD)] = acc

    def start_out(ci, s):
        pltpu.async_copy(out_v.at[s],
                         out_hbm.at[pl.ds(wbase + ci * CH, CH)], sem_out[s])

    def wait_out(s):
        pltpu.make_async_copy(out_v.at[s], out_hbm.at[pl.ds(0, CH)],
                              sem_out[s]).wait()

    stage_idx(0, 0)
    stage_w(0, 0)
    stage_idx(1, 1)
    stage_w(1, 1)
    wait_in(0)
    fire_gathers(0)

    def half(ci, s):
        o = 1 - s

        @pl.when(ci + 1 < nchunks)
        def _():
            wait_in(o)
            fire_gathers(o)

        drain_gathers(s)

        @pl.when(ci + 2 < nchunks)
        def _():
            stage_idx(ci + 2, s)

        @pl.when(ci >= 2)
        def _():
            wait_out(s)

        compute(s)
        start_out(ci, s)

        @pl.when(ci + 2 < nchunks)
        def _():
            stage_w(ci + 2, s)

    def pair(i, carry):
        half(2 * i, 0)
        half(2 * i + 1, 1)
        return carry

    lax.fori_loop(0, nchunks // 2, pair, 0)
    wait_out(0)
    wait_out(1)


def _sc_gather(table, idx, w):
    qtot = idx.shape[0]
    mesh = plsc.VectorSubcoreMesh(core_axis_name="c", subcore_axis_name="s")
    return pl.kernel(
        functools.partial(_sc_gather_body, qtot),
        out_type=jax.ShapeDtypeStruct((qtot, HEADS * HD), jnp.float32),
        mesh=mesh,
        scratch_types=[
            pltpu.VMEM((2, CH, 128), jnp.int32),
            pltpu.VMEM((2, CH, 128), jnp.float32),
            pltpu.VMEM((2, CH, 128, HD), jnp.float32),
            pltpu.VMEM((2, CH, 128), jnp.float32),
            pltpu.SemaphoreType.DMA,
            pltpu.SemaphoreType.DMA,
            pltpu.SemaphoreType.DMA,
            pltpu.SemaphoreType.DMA,
            pltpu.SemaphoreType.DMA,
            pltpu.SemaphoreType.DMA,
        ],
        compiler_params=pltpu.CompilerParams(use_tc_tiling_on_sc=False),
    )(table, idx, w)


def kernel(x, W_off, b_off, W_attn, b_attn, W_val, b_val, W_out, b_out):
    x2 = x.reshape(B, E, NQ)
    ell = np.arange(128)
    p = ell >> 5
    k = (ell >> 3) & 3
    h = ell & 7
    rx = h * (PTS * 2) + p * 2
    Wx = W_off[rx].T
    bx = b_off[rx].reshape(1, 128) - np.float32(0.5)
    Wy = W_off[rx + 1].T
    by = b_off[rx + 1].reshape(1, 128) - np.float32(0.5)
    ra = h * PTS + p
    Wa = W_attn[ra].T
    ba = b_attn[ra].reshape(1, 128)
    kx = (k & 1).astype(np.int32)
    ky = (k >> 1).astype(np.int32)
    kxi = jnp.asarray(kx.reshape(1, 128))
    kyi = jnp.asarray(ky.reshape(1, 128))
    hl = jnp.asarray(h.astype(np.int32).reshape(1, 128))
    ax = jnp.asarray((1.0 - kx).astype(np.float32).reshape(1, 128))
    bxk = jnp.asarray((2.0 * kx - 1.0).astype(np.float32).reshape(1, 128))
    ay = jnp.asarray((1.0 - ky).astype(np.float32).reshape(1, 128))
    byk = jnp.asarray((2.0 * ky - 1.0).astype(np.float32).reshape(1, 128))

    nhalf = 8
    bh = B // nhalf
    qtot_h = bh * NQ
    gpb = NQ // QBLK
    grid = (bh, gpb)
    wspec = pl.BlockSpec((E, 128), lambda b_, g_: (0, 0))
    bspec = pl.BlockSpec((1, 128), lambda b_, g_: (0, 0))
    qspec = pl.BlockSpec((QBLK, 128), lambda b_, g_: (b_ * gpb + g_, 0))
    wv_t = W_val.T
    bv = b_val.reshape(1, 128)
    bo = b_out.reshape(128, 1)

    outs = []
    for hf in range(nhalf):
        xh = lax.slice_in_dim(x2, hf * bh, (hf + 1) * bh, axis=0)
        val, idx, w = pl.pallas_call(
            _prep_body,
            grid=grid,
            in_specs=[
                pl.BlockSpec((1, E, QBLK), lambda b_, g_: (b_, 0, g_)),
                wspec, bspec, wspec, bspec, wspec, bspec, wspec, bspec,
                bspec, bspec, bspec, bspec, bspec, bspec, bspec,
            ],
            out_specs=[qspec, qspec, qspec],
            out_shape=[
                jax.ShapeDtypeStruct((qtot_h, E), jnp.float32),
                jax.ShapeDtypeStruct((qtot_h, 128), jnp.int32),
                jax.ShapeDtypeStruct((qtot_h, 128), jnp.float32),
            ],
        )(xh, wv_t, bv, Wx, bx, Wy, by, Wa, ba,
          kxi, kyi, hl, ax, bxk, ay, byk)

        table = val.reshape(qtot_h * HEADS, HD)
        sampled = _sc_gather(table, idx, w)

        out_h = pl.pallas_call(
            _post_body,
            grid=grid,
            in_specs=[
                qspec,
                pl.BlockSpec((1, E, QBLK), lambda b_, g_: (b_, 0, g_)),
                wspec,
                pl.BlockSpec((E, 1), lambda b_, g_: (0, 0)),
            ],
            out_specs=pl.BlockSpec((1, E, QBLK), lambda b_, g_: (b_, 0, g_)),
            out_shape=jax.ShapeDtypeStruct((bh, E, NQ), jnp.float32),
        )(sampled, xh, W_out, bo)
        outs.append(out_h)
    out = jnp.concatenate(outs, axis=0)
    return out.reshape(B, E, H, W)


# speedup vs baseline: 1.0847x; 1.0214x over previous
import functools
import numpy as np
import jax
import jax.numpy as jnp
from jax import lax
from jax.experimental import pallas as pl
from jax.experimental.pallas import tpu as pltpu
from jax.experimental.pallas import tpu_sc as plsc

B, E, H, W = 8, 128, 128, 128
NQ = H * W
HEADS, PTS, HD = 8, 4, 16
QBLK = 256
QTOT = B * NQ
NC, NS = 2, 16
NW = NC * NS
CH = 16


_DN_T = (((0,), (0,)), ((), ()))


def _prep_body(x_ref, wv_ref, bv_ref, wx_ref, bx_ref, wy_ref, by_ref,
               wa_ref, ba_ref, kxi_ref, kyi_ref, hl_ref,
               ax_ref, bxk_ref, ay_ref, byk_ref,
               val_ref, idx_ref, w_ref):
    b = pl.program_id(0)
    g = pl.program_id(1)
    xb = x_ref[0]
    val_ref[...] = lax.dot_general(xb, wv_ref[...], _DN_T,
                                 preferred_element_type=jnp.float32) + bv_ref[0]
    offx = lax.dot_general(xb, wx_ref[...], _DN_T,
                           preferred_element_type=jnp.float32) + bx_ref[0]
    offy = lax.dot_general(xb, wy_ref[...], _DN_T,
                           preferred_element_type=jnp.float32) + by_ref[0]
    al = lax.dot_general(xb, wa_ref[...], _DN_T,
                         preferred_element_type=jnp.float32) + ba_ref[0]
    m = al
    for s in (32, 64, 96):
        m = jnp.maximum(m, pltpu.roll(al, s, 1))
    ex = jnp.exp(al - m)
    ssum = ex
    for s in (32, 64, 96):
        ssum = ssum + pltpu.roll(ex, s, 1)
    attn = ex / ssum
    i0 = lax.broadcasted_iota(jnp.int32, (QBLK, 128), 0)
    col = i0 & (W - 1)
    rowf = (jnp.where(i0 < W, np.float32(0.0), np.float32(H / (H - 1)))
            + g.astype(jnp.float32) * np.float32((QBLK // W) * H / (H - 1)))
    gx = col.astype(jnp.float32) * np.float32(W / (W - 1)) + offx
    gy = rowf + offy
    x0f = jnp.floor(gx)
    y0f = jnp.floor(gy)
    fx = gx - x0f
    fy = gy - y0f
    xi = x0f.astype(jnp.int32) + kxi_ref[0]
    yi = y0f.astype(jnp.int32) + kyi_ref[0]
    wx = ax_ref[0] + bxk_ref[0] * fx
    wy = ay_ref[0] + byk_ref[0] * fy
    ux = xi.astype(jnp.uint32) < np.uint32(W)
    uy = yi.astype(jnp.uint32) < np.uint32(H)
    valid = ux & uy
    spat = ((yi & (H - 1)) << 7) | (xi & (W - 1))
    idx_ref[...] = (spat << 3) + (b * np.int32(NQ * HEADS) + hl_ref[0])
    w_ref[...] = (attn * wx) * (wy * jnp.where(valid, np.float32(1.0),
                                             np.float32(0.0)))


def _post_body(s_ref, x_ref, wo_ref, bo_ref, out_ref):
    y = lax.dot_general(wo_ref[...], s_ref[...], (((1,), (1,)), ((), ())),
                        preferred_element_type=jnp.float32)
    out_ref[0] = (y + bo_ref[...]) + x_ref[0]


def _sc_gather_body(qtot, table_hbm, idx_hbm, w_hbm, out_hbm,
                    idx_v, w_v, rows_v, out_v,
                    sem_in0, sem_in1, sem_g0, sem_g1, sem_out0, sem_out1):
    cid = lax.axis_index("c")
    sid = lax.axis_index("s")
    wid = sid * NC + cid
    per_w = qtot // NW
    nchunks = per_w // CH
    wbase = wid * per_w
    sem_in = (sem_in0, sem_in1)
    sem_g = (sem_g0, sem_g1)
    sem_out = (sem_out0, sem_out1)

    def stage_idx(ci, s):
        pltpu.async_copy(idx_hbm.at[pl.ds(wbase + ci * CH, CH)],
                         idx_v.at[s], sem_in[s])

    def stage_w(ci, s):
        pltpu.async_copy(w_hbm.at[pl.ds(wbase + ci * CH, CH)],
                         w_v.at[s], sem_in[s])

    def wait_in(s):
        pltpu.make_async_copy(idx_hbm.at[pl.ds(0, CH)], idx_v.at[s],
                              sem_in[s]).wait()
        pltpu.make_async_copy(w_hbm.at[pl.ds(0, CH)], w_v.at[s],
                              sem_in[s]).wait()

    def fire_gathers(s):
        for qi in range(CH):
            pltpu.async_copy(table_hbm.at[idx_v.at[s, qi]],
                             rows_v.at[s, qi], sem_g[s])

    def drain_gathers(s):
        for qi in range(CH):
            pltpu.make_async_copy(table_hbm.at[idx_v.at[s, qi]],
                                  rows_v.at[s, qi], sem_g[s]).wait()

    def compute(s):
        @plsc.parallel_loop(0, CH, unroll=2)
        def _(qi):
            wv = [w_v[s, qi, pl.ds(g * 16, 16)] for g in range(8)]
            for hh in range(HEADS):
                acc = wv[0][hh] * rows_v[s, qi, hh, :]
                for mm in range(1, 16):
                    j = mm * 8 + hh
                    acc = acc + wv[mm // 2][(mm % 2) * 8 + hh] * rows_v[s, qi, j, :]
                out_v[s, qi, pl.ds(hh * HD, HD)] = acc

    def start_out(ci, s):
        pltpu.async_copy(out_v.at[s],
                         out_hbm.at[pl.ds(wbase + ci * CH, CH)], sem_out[s])

    def wait_out(s):
        pltpu.make_async_copy(out_v.at[s], out_hbm.at[pl.ds(0, CH)],
                              sem_out[s]).wait()

    stage_idx(0, 0)
    stage_w(0, 0)
    stage_idx(1, 1)
    stage_w(1, 1)
    wait_in(0)
    fire_gathers(0)

    def half(ci, s):
        o = 1 - s

        @pl.when(ci + 1 < nchunks)
        def _():
            wait_in(o)
            fire_gathers(o)

        drain_gathers(s)

        @pl.when(ci + 2 < nchunks)
        def _():
            stage_idx(ci + 2, s)

        @pl.when(ci >= 2)
        def _():
            wait_out(s)

        compute(s)
        start_out(ci, s)

        @pl.when(ci + 2 < nchunks)
        def _():
            stage_w(ci + 2, s)

    def pair(i, carry):
        half(2 * i, 0)
        half(2 * i + 1, 1)
        return carry

    lax.fori_loop(0, nchunks // 2, pair, 0)
    wait_out(0)
    wait_out(1)


def _sc_gather(table, idx, w):
    qtot = idx.shape[0]
    mesh = plsc.VectorSubcoreMesh(core_axis_name="c", subcore_axis_name="s")
    return pl.kernel(
        functools.partial(_sc_gather_body, qtot),
        out_type=jax.ShapeDtypeStruct((qtot, HEADS * HD), jnp.float32),
        mesh=mesh,
        scratch_types=[
            pltpu.VMEM((2, CH, 128), jnp.int32),
            pltpu.VMEM((2, CH, 128), jnp.float32),
            pltpu.VMEM((2, CH, 128, HD), jnp.float32),
            pltpu.VMEM((2, CH, 128), jnp.float32),
            pltpu.SemaphoreType.DMA,
            pltpu.SemaphoreType.DMA,
            pltpu.SemaphoreType.DMA,
            pltpu.SemaphoreType.DMA,
            pltpu.SemaphoreType.DMA,
            pltpu.SemaphoreType.DMA,
        ],
        compiler_params=pltpu.CompilerParams(use_tc_tiling_on_sc=False),
    )(table, idx, w)


def kernel(x, W_off, b_off, W_attn, b_attn, W_val, b_val, W_out, b_out):
    x2 = x.reshape(B, E, NQ)
    ell = np.arange(128)
    p = ell >> 5
    k = (ell >> 3) & 3
    h = ell & 7
    rx = h * (PTS * 2) + p * 2
    Wx = W_off[rx].T
    bx = b_off[rx].reshape(1, 128) - np.float32(0.5)
    Wy = W_off[rx + 1].T
    by = b_off[rx + 1].reshape(1, 128) - np.float32(0.5)
    ra = h * PTS + p
    Wa = W_attn[ra].T
    ba = b_attn[ra].reshape(1, 128)
    kx = (k & 1).astype(np.int32)
    ky = (k >> 1).astype(np.int32)
    kxi = jnp.asarray(kx.reshape(1, 128))
    kyi = jnp.asarray(ky.reshape(1, 128))
    hl = jnp.asarray(h.astype(np.int32).reshape(1, 128))
    ax = jnp.asarray((1.0 - kx).astype(np.float32).reshape(1, 128))
    bxk = jnp.asarray((2.0 * kx - 1.0).astype(np.float32).reshape(1, 128))
    ay = jnp.asarray((1.0 - ky).astype(np.float32).reshape(1, 128))
    byk = jnp.asarray((2.0 * ky - 1.0).astype(np.float32).reshape(1, 128))

    nhalf = 8
    bh = B // nhalf
    qtot_h = bh * NQ
    gpb = NQ // QBLK
    grid = (bh, gpb)
    wspec = pl.BlockSpec((E, 128), lambda b_, g_: (0, 0))
    bspec = pl.BlockSpec((1, 128), lambda b_, g_: (0, 0))
    qspec = pl.BlockSpec((QBLK, 128), lambda b_, g_: (b_ * gpb + g_, 0))
    wv_t = W_val.T
    bv = b_val.reshape(1, 128)
    bo = b_out.reshape(128, 1)

    outs = []
    for hf in range(nhalf):
        xh = lax.slice_in_dim(x2, hf * bh, (hf + 1) * bh, axis=0)
        val, idx, w = pl.pallas_call(
            _prep_body,
            grid=grid,
            in_specs=[
                pl.BlockSpec((1, E, QBLK), lambda b_, g_: (b_, 0, g_)),
                wspec, bspec, wspec, bspec, wspec, bspec, wspec, bspec,
                bspec, bspec, bspec, bspec, bspec, bspec, bspec,
            ],
            out_specs=[qspec, qspec, qspec],
            out_shape=[
                jax.ShapeDtypeStruct((qtot_h, E), jnp.float32),
                jax.ShapeDtypeStruct((qtot_h, 128), jnp.int32),
                jax.ShapeDtypeStruct((qtot_h, 128), jnp.float32),
            ],
        )(xh, wv_t, bv, Wx, bx, Wy, by, Wa, ba,
          kxi, kyi, hl, ax, bxk, ay, byk)

        table = val.reshape(qtot_h * HEADS, HD)
        sampled = _sc_gather(table, idx, w)

        out_h = pl.pallas_call(
            _post_body,
            grid=grid,
            in_specs=[
                qspec,
                pl.BlockSpec((1, E, QBLK), lambda b_, g_: (b_, 0, g_)),
                wspec,
                pl.BlockSpec((E, 1), lambda b_, g_: (0, 0)),
            ],
            out_specs=pl.BlockSpec((1, E, QBLK), lambda b_, g_: (b_, 0, g_)),
            out_shape=jax.ShapeDtypeStruct((bh, E, NQ), jnp.float32),
        )(sampled, xh, W_out, bo)
        outs.append(out_h)
    out = jnp.concatenate(outs, axis=0)
    return out.reshape(B, E, H, W)
